# baseline (speedup 1.0000x reference)
import functools

import jax
import jax.numpy as jnp
from jax import lax
from jax.experimental import pallas as pl
from jax.experimental.pallas import tpu as pltpu
from jax.experimental.pallas import tpu_sc as plsc

_B = 16
_S = 512 * 512
_N = _B * _S
_NC = 2
_NS = 16
_NW = _NC * _NS
_L = 16
_ROWS = 16

_BSC = 7
_T1 = 3
_RPW = _BSC * 512 // _NW
_NCH = _RPW // _ROWS


def _sc_partials(inp4d, tgt3d):
    mesh = plsc.VectorSubcoreMesh(core_axis_name="c", subcore_axis_name="s")

    @functools.partial(
        pl.kernel,
        mesh=mesh,
        out_type=jax.ShapeDtypeStruct((_NW, 4, _L), jnp.float32),
        scratch_types=[
            pltpu.VMEM((2, _ROWS, 512), jnp.float32),
            pltpu.VMEM((2, _ROWS, 512), jnp.float32),
            pltpu.VMEM((2, _ROWS, 512), jnp.int32),
            pltpu.VMEM((4, _L), jnp.float32),
            pltpu.SemaphoreType.DMA,
            pltpu.SemaphoreType.DMA,
            pltpu.SemaphoreType.DMA,
            pltpu.SemaphoreType.DMA,
            pltpu.SemaphoreType.DMA,
            pltpu.SemaphoreType.DMA,
        ],
    )
    def k(inp_hbm, tgt_hbm, out_hbm, x0_v, x1_v, t_v, o_v,
          s00, s01, s02, s10, s11, s12):
        sems = ((s00, s01, s02), (s10, s11, s12))
        wid = lax.axis_index("s") * _NC + lax.axis_index("c")
        g0 = _T1 * 512 + wid * _RPW

        def issue(ci, buf):
            g = g0 + ci * _ROWS
            b = g // 512
            r0 = g % 512
            pltpu.async_copy(inp_hbm.at[b, 0, pl.ds(r0, _ROWS), :],
                             x0_v.at[buf], sems[buf][0])
            pltpu.async_copy(inp_hbm.at[b, 1, pl.ds(r0, _ROWS), :],
                             x1_v.at[buf], sems[buf][1])
            pltpu.async_copy(tgt_hbm.at[b, pl.ds(r0, _ROWS), :],
                             t_v.at[buf], sems[buf][2])

        def wait_buf(buf):
            pltpu.make_async_copy(inp_hbm.at[0, 0, pl.ds(0, _ROWS), :],
                                  x0_v.at[buf], sems[buf][0]).wait()
            pltpu.make_async_copy(inp_hbm.at[0, 1, pl.ds(0, _ROWS), :],
                                  x1_v.at[buf], sems[buf][1]).wait()
            pltpu.make_async_copy(tgt_hbm.at[0, pl.ds(0, _ROWS), :],
                                  t_v.at[buf], sems[buf][2]).wait()

        def compute(buf, acc):
            def body(i, a):
                ac, atc, at = a
                r = i // 4
                q = (i % 4) * 128
                for j in range(8):
                    off = q + j * _L
                    x0 = x0_v[buf, r, pl.ds(off, _L)]
                    x1 = x1_v[buf, r, pl.ds(off, _L)]
                    tt = t_v[buf, r, pl.ds(off, _L)]
                    e = jnp.exp(x1 - x0)
                    cc = 1.0 / (1.0 + e) - 0.5
                    tf = tt.astype(jnp.float32)
                    ac = ac + cc
                    atc = atc + tf * cc
                    at = at + tf
                return (ac, atc, at)
            return lax.fori_loop(0, _ROWS * 4, body, acc)

        issue(0, 0)
        zeros = jnp.zeros((_L,), jnp.float32)
        acc = (zeros, zeros, zeros)

        def outer(g, acc):
            wait_buf(0)
            issue(2 * g + 1, 1)
            acc = compute(0, acc)
            wait_buf(1)

            @pl.when(2 * g + 2 < _NCH)
            def _():
                issue(2 * g + 2, 0)

            acc = compute(1, acc)
            return acc

        acc = lax.fori_loop(0, _NCH // 2, outer, acc)
        if _NCH % 2:
            wait_buf(0)
            acc = compute(0, acc)
        acc_c, acc_tc, acc_t = acc
        o_v[0, :] = acc_c
        o_v[1, :] = acc_tc
        o_v[2, :] = acc_t
        o_v[3, :] = jnp.zeros((_L,), jnp.float32)
        pltpu.sync_copy(o_v, out_hbm.at[wid])

    return k(inp4d, tgt3d)


def _tc_body(x_ref, t_ref, o_ref):
    k = pl.program_id(0)
    x0 = x_ref[0, 0, :, :]
    x1 = x_ref[0, 1, :, :]
    a = 1.0 / (1.0 + jnp.exp(x1 - x0))
    c = a - 0.5
    tf = t_ref[0, :, :].astype(jnp.float32)
    pc = jnp.sum(c.reshape(256, 8, 128), axis=0)
    ptc = jnp.sum((tf * c).reshape(256, 8, 128), axis=0)
    pt = jnp.sum(tf.reshape(256, 8, 128), axis=0)
    part = jnp.stack([pc, ptc, pt])

    @pl.when(k == 0)
    def _():
        o_ref[...] = jnp.zeros_like(o_ref)

    o_ref[...] += part


def _tc_partials(inp4d, tgt3d, b0, nb):
    return pl.pallas_call(
        _tc_body,
        grid=(nb,),
        in_specs=[
            pl.BlockSpec((1, 2, 512, 512), lambda k: (k + b0, 0, 0, 0)),
            pl.BlockSpec((1, 512, 512), lambda k: (k + b0, 0, 0)),
        ],
        out_specs=pl.BlockSpec((3, 8, 128), lambda k: (0, 0, 0)),
        out_shape=jax.ShapeDtypeStruct((3, 8, 128), jnp.float32),
    )(inp4d, tgt3d)


def kernel(inputs, target):
    p1 = _tc_partials(inputs, target, 0, _T1)
    ib, tb, p1b = lax.optimization_barrier((inputs, target, p1))
    p_sc = _sc_partials(ib, tb)
    p2 = _tc_partials(ib, tb, _T1 + _BSC, _B - _T1 - _BSC)
    p_tc = p1b + p2
    sum_c = jnp.sum(p_sc[:, 0, :]) + jnp.sum(p_tc[0])
    sum_tc = jnp.sum(p_sc[:, 1, :]) + jnp.sum(p_tc[1])
    sum_t = jnp.sum(p_sc[:, 2, :]) + jnp.sum(p_tc[2])
    n1 = sum_t
    n0 = jnp.float32(_N) - sum_t
    e1 = sum_tc
    e0 = sum_c - sum_tc
    det = e0 * n1 - e1 * n0
    return -jnp.log(jnp.abs(det) + 0.001)


# speedup vs baseline: 1.0651x; 1.0651x over previous
import functools

import jax
import jax.numpy as jnp
from jax import lax
from jax.experimental import pallas as pl
from jax.experimental.pallas import tpu as pltpu
from jax.experimental.pallas import tpu_sc as plsc

_B = 16
_S = 512 * 512
_N = _B * _S
_NC = 2
_NS = 16
_NW = _NC * _NS
_L = 16
_ROWS = 16

_BSC = 7
_RPW = _BSC * 512 // _NW
_NCH = _RPW // _ROWS


def _sc_partials(inp4d, tgt3d):
    mesh = plsc.VectorSubcoreMesh(core_axis_name="c", subcore_axis_name="s")

    @functools.partial(
        pl.kernel,
        mesh=mesh,
        out_type=jax.ShapeDtypeStruct((_NW, 4, _L), jnp.float32),
        scratch_types=[
            pltpu.VMEM((4, _ROWS, 512), jnp.float32),
            pltpu.VMEM((4, _ROWS, 512), jnp.float32),
            pltpu.VMEM((4, _ROWS, 512), jnp.int32),
            pltpu.VMEM((4, _L), jnp.float32),
            pltpu.SemaphoreType.DMA,
            pltpu.SemaphoreType.DMA,
            pltpu.SemaphoreType.DMA,
            pltpu.SemaphoreType.DMA,
        ],
    )
    def k(inp_hbm, tgt_hbm, out_hbm, x0_v, x1_v, t_v, o_v,
          s0, s1, s2, s3):
        sems = (s0, s1, s2, s3)
        wid = lax.axis_index("s") * _NC + lax.axis_index("c")
        g0 = wid * _RPW

        def issue(ci, buf):
            g = g0 + ci * _ROWS
            b = g // 512
            r0 = g % 512
            pltpu.async_copy(inp_hbm.at[b, 0, pl.ds(r0, _ROWS), :],
                             x0_v.at[buf], sems[buf])
            pltpu.async_copy(inp_hbm.at[b, 1, pl.ds(r0, _ROWS), :],
                             x1_v.at[buf], sems[buf])
            pltpu.async_copy(tgt_hbm.at[b, pl.ds(r0, _ROWS), :],
                             t_v.at[buf], sems[buf])

        def wait_buf(buf):
            for _ in range(3):
                pltpu.make_async_copy(tgt_hbm.at[0, pl.ds(0, _ROWS), :],
                                      t_v.at[buf], sems[buf]).wait()

        def compute(buf, acc):
            def body(i, a):
                ac, atc, at = a
                r = i // 4
                q = (i % 4) * 128
                for j in range(8):
                    off = q + j * _L
                    x0 = x0_v[buf, r, pl.ds(off, _L)]
                    x1 = x1_v[buf, r, pl.ds(off, _L)]
                    tt = t_v[buf, r, pl.ds(off, _L)]
                    e = jnp.exp(x1 - x0)
                    cc = 1.0 / (1.0 + e) - 0.5
                    tf = tt.astype(jnp.float32)
                    ac = ac + cc
                    atc = atc + tf * cc
                    at = at + tf
                return (ac, atc, at)
            return lax.fori_loop(0, _ROWS * 4, body, acc)

        for p in range(min(3, _NCH)):
            issue(p, p)
        zeros = jnp.zeros((_L,), jnp.float32)
        acc = (zeros, zeros, zeros)
        for ci in range(_NCH):
            wait_buf(ci % 4)
            if ci + 3 < _NCH:
                issue(ci + 3, (ci + 3) % 4)
            acc = compute(ci % 4, acc)
        acc_c, acc_tc, acc_t = acc
        o_v[0, :] = acc_c
        o_v[1, :] = acc_tc
        o_v[2, :] = acc_t
        o_v[3, :] = jnp.zeros((_L,), jnp.float32)
        pltpu.sync_copy(o_v, out_hbm.at[wid])

    return k(inp4d, tgt3d)


def _tc_body(x_ref, t_ref, o_ref):
    k = pl.program_id(0)
    x0 = x_ref[0, 0, :, :]
    x1 = x_ref[0, 1, :, :]
    a = 1.0 / (1.0 + jnp.exp(x1 - x0))
    c = a - 0.5
    tf = t_ref[0, :, :].astype(jnp.float32)
    pc = jnp.sum(c.reshape(256, 8, 128), axis=0)
    ptc = jnp.sum((tf * c).reshape(256, 8, 128), axis=0)
    pt = jnp.sum(tf.reshape(256, 8, 128), axis=0)
    part = jnp.stack([pc, ptc, pt])

    @pl.when(k == 0)
    def _():
        o_ref[...] = jnp.zeros_like(o_ref)

    o_ref[...] += part


def _tc_partials(inp4d, tgt3d, b0, nb):
    return pl.pallas_call(
        _tc_body,
        grid=(nb,),
        in_specs=[
            pl.BlockSpec((1, 2, 512, 512), lambda k: (k + b0, 0, 0, 0)),
            pl.BlockSpec((1, 512, 512), lambda k: (k + b0, 0, 0)),
        ],
        out_specs=pl.BlockSpec((3, 8, 128), lambda k: (0, 0, 0)),
        out_shape=jax.ShapeDtypeStruct((3, 8, 128), jnp.float32),
    )(inp4d, tgt3d)


def kernel(inputs, target):
    p_tc = _tc_partials(inputs, target, _BSC, _B - _BSC)
    p_sc = _sc_partials(inputs, target)
    sum_c = jnp.sum(p_sc[:, 0, :]) + jnp.sum(p_tc[0])
    sum_tc = jnp.sum(p_sc[:, 1, :]) + jnp.sum(p_tc[1])
    sum_t = jnp.sum(p_sc[:, 2, :]) + jnp.sum(p_tc[2])
    n1 = sum_t
    n0 = jnp.float32(_N) - sum_t
    e1 = sum_tc
    e0 = sum_c - sum_tc
    det = e0 * n1 - e1 * n0
    return -jnp.log(jnp.abs(det) + 0.001)


# speedup vs baseline: 1.0861x; 1.0197x over previous
import functools

import jax
import jax.numpy as jnp
from jax import lax
from jax.experimental import pallas as pl
from jax.experimental.pallas import tpu as pltpu
from jax.experimental.pallas import tpu_sc as plsc

_B = 16
_S = 512 * 512
_N = _B * _S
_NC = 2
_NS = 16
_NW = _NC * _NS
_L = 16
_ROWS = 16

_BSC = 7
_RPW = _BSC * 512 // _NW
_NCH = _RPW // _ROWS


def _sc_partials(inp4d, tgt3d):
    mesh = plsc.VectorSubcoreMesh(core_axis_name="c", subcore_axis_name="s")

    @functools.partial(
        pl.kernel,
        mesh=mesh,
        out_type=jax.ShapeDtypeStruct((_NW, 4, _L), jnp.float32),
        scratch_types=[
            pltpu.VMEM((2, 2, _ROWS, 512), jnp.float32),
            pltpu.VMEM((2, _ROWS, 512), jnp.int32),
            pltpu.VMEM((4, _L), jnp.float32),
            pltpu.SemaphoreType.DMA,
            pltpu.SemaphoreType.DMA,
            pltpu.SemaphoreType.DMA,
            pltpu.SemaphoreType.DMA,
            pltpu.SemaphoreType.DMA,
            pltpu.SemaphoreType.DMA,
        ],
    )
    def k(inp_hbm, tgt_hbm, out_hbm, x_v, t_v, o_v,
          s00, s01, s02, s10, s11, s12):
        sems = ((s00, s01, s02), (s10, s11, s12))
        wid = lax.axis_index("s") * _NC + lax.axis_index("c")
        g0 = wid * _RPW

        def issue(ci, buf):
            g = g0 + ci * _ROWS
            b = g // 512
            r0 = g % 512
            pltpu.async_copy(inp_hbm.at[b, :, pl.ds(r0, _ROWS), :],
                             x_v.at[buf], sems[buf][0])
            pltpu.async_copy(tgt_hbm.at[b, pl.ds(r0, _ROWS), :],
                             t_v.at[buf], sems[buf][2])

        def wait_buf(buf):
            pltpu.make_async_copy(inp_hbm.at[0, :, pl.ds(0, _ROWS), :],
                                  x_v.at[buf], sems[buf][0]).wait()
            pltpu.make_async_copy(tgt_hbm.at[0, pl.ds(0, _ROWS), :],
                                  t_v.at[buf], sems[buf][2]).wait()

        def compute(buf, acc):
            def body(i, a):
                ac, atc, at = a
                r = i // 4
                q = (i % 4) * 128
                for j in range(8):
                    off = q + j * _L
                    x0 = x_v[buf, 0, r, pl.ds(off, _L)]
                    x1 = x_v[buf, 1, r, pl.ds(off, _L)]
                    tt = t_v[buf, r, pl.ds(off, _L)]
                    e = jnp.exp(x1 - x0)
                    cc = 1.0 / (1.0 + e) - 0.5
                    tf = tt.astype(jnp.float32)
                    ac = ac + cc
                    atc = atc + tf * cc
                    at = at + tf
                return (ac, atc, at)
            return lax.fori_loop(0, _ROWS * 4, body, acc)

        issue(0, 0)
        zeros = jnp.zeros((_L,), jnp.float32)
        acc = (zeros, zeros, zeros)

        def outer(g, acc):
            wait_buf(0)
            issue(2 * g + 1, 1)
            acc = compute(0, acc)
            wait_buf(1)

            @pl.when(2 * g + 2 < _NCH)
            def _():
                issue(2 * g + 2, 0)

            acc = compute(1, acc)
            return acc

        acc = lax.fori_loop(0, _NCH // 2, outer, acc)
        if _NCH % 2:
            wait_buf(0)
            acc = compute(0, acc)
        acc_c, acc_tc, acc_t = acc
        o_v[0, :] = acc_c
        o_v[1, :] = acc_tc
        o_v[2, :] = acc_t
        o_v[3, :] = jnp.zeros((_L,), jnp.float32)
        pltpu.sync_copy(o_v, out_hbm.at[wid])

    return k(inp4d, tgt3d)


def _tc_body(x_ref, t_ref, o_ref):
    k = pl.program_id(0)
    x0 = x_ref[0, 0, :, :]
    x1 = x_ref[0, 1, :, :]
    a = 1.0 / (1.0 + jnp.exp(x1 - x0))
    c = a - 0.5
    tf = t_ref[0, :, :].astype(jnp.float32)
    pc = jnp.sum(c.reshape(256, 8, 128), axis=0)
    ptc = jnp.sum((tf * c).reshape(256, 8, 128), axis=0)
    pt = jnp.sum(tf.reshape(256, 8, 128), axis=0)
    part = jnp.stack([pc, ptc, pt])

    @pl.when(k == 0)
    def _():
        o_ref[...] = jnp.zeros_like(o_ref)

    o_ref[...] += part


def _tc_partials(inp4d, tgt3d, b0, nb):
    return pl.pallas_call(
        _tc_body,
        grid=(nb,),
        in_specs=[
            pl.BlockSpec((1, 2, 512, 512), lambda k: (k + b0, 0, 0, 0)),
            pl.BlockSpec((1, 512, 512), lambda k: (k + b0, 0, 0)),
        ],
        out_specs=pl.BlockSpec((3, 8, 128), lambda k: (0, 0, 0)),
        out_shape=jax.ShapeDtypeStruct((3, 8, 128), jnp.float32),
    )(inp4d, tgt3d)


def kernel(inputs, target):
    p_tc = _tc_partials(inputs, target, _BSC, _B - _BSC)
    p_sc = _sc_partials(inputs, target)
    sum_c = jnp.sum(p_sc[:, 0, :]) + jnp.sum(p_tc[0])
    sum_tc = jnp.sum(p_sc[:, 1, :]) + jnp.sum(p_tc[1])
    sum_t = jnp.sum(p_sc[:, 2, :]) + jnp.sum(p_tc[2])
    n1 = sum_t
    n0 = jnp.float32(_N) - sum_t
    e1 = sum_tc
    e0 = sum_c - sum_tc
    det = e0 * n1 - e1 * n0
    return -jnp.log(jnp.abs(det) + 0.001)
